# baseline (speedup 1.0000x reference)
import functools
import math

import jax
import jax.numpy as jnp
from jax import lax
from jax.experimental import pallas as pl
from jax.experimental.pallas import tpu as pltpu
from jax.experimental.pallas import tpu_sc as plsc

N = 10000
E = 160000
D_IN = 256
D_H = 512
L_LAYERS = 4
ALPHA = 0.1
THETA = 0.5

NCHUNK = 4
CW = D_H // NCHUNK
NC, NS = 2, 16
NLOC = N // NC
ACCROWS = NLOC + 8
EDGES_PER_SUB = E // NS
BATCH = 80
NBATCH = EDGES_PER_SUB // BATCH
RING = 5
ZR = 8
ZBLK = 312 // ZR

R = 1000
GRID = N // R


def _sc_body(h4, src4, dst2, cnt, agg, zbuf, src_i, dst_i, cnt_v,
             r0, r1, r2, r3, r4, acc, gsem, ssem, fsem):
    rows = [r0, r1, r2, r3, r4]
    c = lax.axis_index("c")
    s = lax.axis_index("s")

    zero16 = jnp.zeros((16,), jnp.float32)

    def _zrow(i, carry):
        for j in range(CW // 16):
            zbuf[i, pl.ds(j * 16, 16)] = zero16
        return carry

    lax.fori_loop(0, ZR, _zrow, 0)

    row0 = s * 312
    nblk = jnp.where(s == NS - 1, ZBLK + 1, ZBLK)
    pltpu.sync_copy(dst2.at[c, s], dst_i)
    pltpu.sync_copy(cnt.at[c], cnt_v)
    n_edges = cnt_v[pl.ds(s, 16)][0]
    nb = lax.div(n_edges + BATCH - 1, BATCH)

    for cc in range(NCHUNK):
        pltpu.sync_copy(src4.at[cc, c, s], src_i)

        nzblk = jnp.where(s == NS - 1, ZBLK + 2, ZBLK)

        def _zacc(i, carry):
            pltpu.async_copy(zbuf, acc.at[pl.ds(row0 + i * ZR, ZR)], fsem)
            return carry

        lax.fori_loop(0, nzblk, _zacc, 0)

        def _zacc_drain(i, carry):
            pltpu.make_async_copy(
                zbuf, acc.at[pl.ds(row0, ZR)], fsem).wait()
            return carry

        lax.fori_loop(0, nzblk, _zacc_drain, 0)
        plsc.subcore_barrier()

        @pl.when(nb > 0)
        def _():
            pltpu.async_copy(h4.at[src_i.at[0]], rows[0], gsem)

        def _step(bb, carry):
            for k in range(RING):
                b = bb * RING + k
                live = b < nb

                @pl.when(live)
                def _():
                    pltpu.make_async_copy(
                        h4.at[src_i.at[b]], rows[k], gsem).wait()

                @pl.when(live & (b >= 4))
                def _():
                    pltpu.make_async_copy(
                        rows[(k - 4) % RING],
                        acc.at[dst_i.at[b - 4]], ssem).wait()

                @pl.when(b + 1 < nb)
                def _():
                    pltpu.async_copy(
                        h4.at[src_i.at[b + 1]], rows[(k + 1) % RING], gsem)

                @pl.when(live)
                def _():
                    pltpu.async_copy(
                        rows[k], acc.at[dst_i.at[b]], ssem, add=True)
            return carry

        lax.fori_loop(0, lax.div(nb + RING - 1, RING), _step, 0)
        for t in range(4):
            @pl.when(nb >= 4 - t)
            def _():
                pltpu.make_async_copy(
                    rows[t], acc.at[dst_i.at[0]], ssem).wait()
        plsc.subcore_barrier()

        def _flush(i, carry):
            pltpu.async_copy(
                acc.at[pl.ds(row0 + i * ZR, ZR)],
                agg.at[pl.ds(cc * N + c * NLOC + row0 + i * ZR, ZR)], fsem)
            return carry

        lax.fori_loop(0, nblk, _flush, 0)

        def _flush_drain(i, carry):
            pltpu.make_async_copy(
                acc.at[pl.ds(row0, ZR)],
                agg.at[pl.ds(cc * N, ZR)], fsem).wait()
            return carry

        lax.fori_loop(0, nblk, _flush_drain, 0)
        if cc < NCHUNK - 1:
            plsc.subcore_barrier()


def _make_sc_scatter():
    mesh = plsc.VectorSubcoreMesh(
        core_axis_name="c", subcore_axis_name="s", num_cores=NC,
        num_subcores=NS)

    return pl.kernel(
        _sc_body,
        out_type=jax.ShapeDtypeStruct((NCHUNK * N, CW), jnp.float32),
        mesh=mesh,
        scratch_types=[
            pltpu.VMEM((ZR, CW), jnp.float32),
            pltpu.VMEM((NBATCH, BATCH), jnp.int32),
            pltpu.VMEM((NBATCH, BATCH), jnp.int32),
            pltpu.VMEM((2 * NS,), jnp.int32),
            pltpu.VMEM((BATCH, CW), jnp.float32),
            pltpu.VMEM((BATCH, CW), jnp.float32),
            pltpu.VMEM((BATCH, CW), jnp.float32),
            pltpu.VMEM((BATCH, CW), jnp.float32),
            pltpu.VMEM((BATCH, CW), jnp.float32),
            pltpu.VMEM_SHARED((ACCROWS, CW), jnp.float32),
            pltpu.SemaphoreType.DMA,
            pltpu.SemaphoreType.DMA,
            pltpu.SemaphoreType.DMA,
        ],
    )


_sc_scatter = _make_sc_scatter()


def _lin1_body(x_ref, w_ref, b_ref, out_ref):
    out_ref[...] = jnp.dot(x_ref[...], w_ref[...],
                           preferred_element_type=jnp.float32) + b_ref[...]


_lin1 = pl.pallas_call(
    _lin1_body,
    grid=(GRID,),
    in_specs=[
        pl.BlockSpec((R, D_IN), lambda r: (r, 0)),
        pl.BlockSpec((D_IN, D_H), lambda r: (0, 0)),
        pl.BlockSpec((1, D_H), lambda r: (0, 0)),
    ],
    out_specs=pl.BlockSpec((R, D_H), lambda r: (r, 0)),
    out_shape=jax.ShapeDtypeStruct((N, D_H), jnp.float32),
)


def _layer_body(agg_ref, x0_ref, w_ref, out_ref, *, beta):
    agg = jnp.concatenate([agg_ref[c] for c in range(NCHUNK)], axis=-1)
    hh = (1.0 - ALPHA) * agg + ALPHA * x0_ref[...]
    out_ref[...] = (1.0 - beta) * hh + beta * jnp.dot(
        hh, w_ref[...], preferred_element_type=jnp.float32)


def _make_layer(beta):
    return pl.pallas_call(
        functools.partial(_layer_body, beta=beta),
        grid=(GRID,),
        in_specs=[
            pl.BlockSpec((NCHUNK, R, CW), lambda r: (0, r, 0)),
            pl.BlockSpec((R, D_H), lambda r: (r, 0)),
            pl.BlockSpec((D_H, D_H), lambda r: (0, 0)),
        ],
        out_specs=pl.BlockSpec((R, D_H), lambda r: (r, 0)),
        out_shape=jax.ShapeDtypeStruct((N, D_H), jnp.float32),
    )


_layers = [_make_layer(float(math.log(THETA / (i + 1) + 1.0)))
           for i in range(L_LAYERS)]


def _fuse_body(h0_ref, h1_ref, h2_ref, h3_ref, wf_ref, bf_ref, w2_ref, b2_ref,
               out_ref):
    hs = [h0_ref, h1_ref, h2_ref, h3_ref]
    acc = jnp.broadcast_to(bf_ref[...], (R, D_H)).astype(jnp.float32)
    for i in range(L_LAYERS):
        acc = acc + jnp.dot(hs[i][...], wf_ref[i],
                            preferred_element_type=jnp.float32)
    out_ref[...] = jnp.dot(acc, w2_ref[...],
                           preferred_element_type=jnp.float32) + b2_ref[...]


_fuse = pl.pallas_call(
    _fuse_body,
    grid=(GRID,),
    in_specs=[
        pl.BlockSpec((R, D_H), lambda r: (r, 0)),
        pl.BlockSpec((R, D_H), lambda r: (r, 0)),
        pl.BlockSpec((R, D_H), lambda r: (r, 0)),
        pl.BlockSpec((R, D_H), lambda r: (r, 0)),
        pl.BlockSpec((L_LAYERS, D_H, D_H), lambda r: (0, 0, 0)),
        pl.BlockSpec((1, D_H), lambda r: (0, 0)),
        pl.BlockSpec((D_H, D_H), lambda r: (0, 0)),
        pl.BlockSpec((1, D_H), lambda r: (0, 0)),
    ],
    out_specs=pl.BlockSpec((R, D_H), lambda r: (r, 0)),
    out_shape=jax.ShapeDtypeStruct((N, D_H), jnp.float32),
)


def kernel(x, edge_index, w_lin1, b_lin1, conv_ws, w_fuse, b_fuse, w_lin2,
           b_lin2):
    src = edge_index[0]
    dst = edge_index[1]
    cap = EDGES_PER_SUB
    h2 = (dst >= NLOC).reshape(NS, cap)
    src_r = src.reshape(NS, cap)
    dst_r = jnp.where(h2, dst.reshape(NS, cap) - NLOC, dst.reshape(NS, cap))
    c1 = jnp.cumsum(h2.astype(jnp.int32), axis=1)
    c0 = jnp.arange(1, cap + 1, dtype=jnp.int32)[None, :] - c1
    tilebase = jnp.arange(NS, dtype=jnp.int32)[:, None] * cap
    oob = NS * cap
    p0 = jnp.where(h2, oob, tilebase + c0 - 1).reshape(-1)
    p1 = jnp.where(h2, tilebase + c1 - 1, oob).reshape(-1)
    src_f = src_r.reshape(-1)
    dst_f = dst_r.reshape(-1)
    psrc0 = jnp.zeros((NS * cap,), jnp.int32).at[p0].set(
        src_f, mode="drop", unique_indices=True)
    pdst0 = jnp.full((NS * cap,), NLOC, jnp.int32).at[p0].set(
        dst_f, mode="drop", unique_indices=True)
    psrc1 = jnp.zeros((NS * cap,), jnp.int32).at[p1].set(
        src_f, mode="drop", unique_indices=True)
    pdst1 = jnp.full((NS * cap,), NLOC, jnp.int32).at[p1].set(
        dst_f, mode="drop", unique_indices=True)
    psrc = jnp.stack([psrc0, psrc1])
    src4 = (psrc[None, :, :] * NCHUNK
            + jnp.arange(NCHUNK, dtype=jnp.int32)[:, None, None])
    src4 = src4.reshape(NCHUNK, NC, NS, NBATCH, BATCH)
    dst2 = jnp.stack([pdst0, pdst1]).reshape(NC, NS, NBATCH, BATCH)
    n1 = c1[:, -1]
    cnt = jnp.stack([cap - n1, n1]).astype(jnp.int32)
    cnt = jnp.pad(cnt, ((0, 0), (0, NS)))

    x0 = _lin1(x, w_lin1, b_lin1.reshape(1, D_H))
    h = x0
    feats = []
    for i in range(L_LAYERS):
        agg = _sc_scatter(h.reshape(NCHUNK * N, CW), src4, dst2, cnt)
        h = _layers[i](agg.reshape(NCHUNK, N, CW), x0, conv_ws[i])
        feats.append(h)

    return _fuse(feats[0], feats[1], feats[2], feats[3],
                 w_fuse.reshape(L_LAYERS, D_H, D_H),
                 b_fuse.reshape(1, D_H),
                 w_lin2, b_lin2.reshape(1, D_H))


# speedup vs baseline: 1.7477x; 1.7477x over previous
import functools
import math

import jax
import jax.numpy as jnp
from jax import lax
from jax.experimental import pallas as pl
from jax.experimental.pallas import tpu as pltpu
from jax.experimental.pallas import tpu_sc as plsc

N = 10000
E = 160000
D_IN = 256
D_H = 512
L_LAYERS = 4
ALPHA = 0.1
THETA = 0.5

NCHUNK = 4
CW = D_H // NCHUNK
NC, NS = 2, 16
NLOC = N // NC
ACCROWS = NLOC + 8
BATCH = 80
NBATCH = 125
EPS = E // NS
RING = 5
ZR = 8
ZBLK = 312 // ZR

R = 1000
GRID = N // R


def _sc_body(h4, src4, dst2, agg, zbuf, src_i, dst_i, r0, r1, r2, r3, r4,
             acc, gsem, ssem, fsem):
    rows = [r0, r1, r2, r3, r4]
    c = lax.axis_index("c")
    s = lax.axis_index("s")

    zero16 = jnp.zeros((16,), jnp.float32)

    def _zrow(i, carry):
        for j in range(CW // 16):
            zbuf[i, pl.ds(j * 16, 16)] = zero16
        return carry

    lax.fori_loop(0, ZR, _zrow, 0)

    row0 = s * 312
    nblk = jnp.where(s == NS - 1, 40, ZBLK)
    nzblk = jnp.where(s == NS - 1, 41, ZBLK)
    pltpu.sync_copy(dst2.at[c, s], dst_i)

    for cc in range(NCHUNK):
        pltpu.sync_copy(src4.at[cc, s], src_i)

        def _zacc(i, carry):
            pltpu.async_copy(zbuf, acc.at[pl.ds(row0 + i * ZR, ZR)], fsem)
            return carry

        lax.fori_loop(0, nzblk, _zacc, 0)

        def _zacc_drain(i, carry):
            pltpu.make_async_copy(
                zbuf, acc.at[pl.ds(row0, ZR)], fsem).wait()
            return carry

        lax.fori_loop(0, nzblk, _zacc_drain, 0)
        plsc.subcore_barrier()

        pltpu.async_copy(h4.at[src_i.at[0]], rows[0], gsem)

        def _step(bb, carry):
            for k in range(RING):
                b = bb * RING + k
                live = b < NBATCH

                @pl.when(live)
                def _():
                    pltpu.make_async_copy(
                        h4.at[src_i.at[b]], rows[k], gsem).wait()

                @pl.when(live & (b >= 4))
                def _():
                    pltpu.make_async_copy(
                        rows[(k - 4) % RING],
                        acc.at[dst_i.at[b - 4]], ssem).wait()

                @pl.when(b + 1 < NBATCH)
                def _():
                    pltpu.async_copy(
                        h4.at[src_i.at[b + 1]], rows[(k + 1) % RING], gsem)

                @pl.when(live)
                def _():
                    pltpu.async_copy(
                        rows[k], acc.at[dst_i.at[b]], ssem, add=True)
            return carry

        lax.fori_loop(0, (NBATCH + RING - 1) // RING, _step, 0)
        for t in range(4):
            pltpu.make_async_copy(
                rows[t], acc.at[dst_i.at[0]], ssem).wait()
        plsc.subcore_barrier()

        def _flush(i, carry):
            pltpu.async_copy(
                acc.at[pl.ds(row0 + i * ZR, ZR)],
                agg.at[pl.ds(cc * N + c * NLOC + row0 + i * ZR, ZR)], fsem)
            return carry

        lax.fori_loop(0, nblk, _flush, 0)

        def _flush_drain(i, carry):
            pltpu.make_async_copy(
                acc.at[pl.ds(row0, ZR)],
                agg.at[pl.ds(cc * N, ZR)], fsem).wait()
            return carry

        lax.fori_loop(0, nblk, _flush_drain, 0)
        if cc < NCHUNK - 1:
            plsc.subcore_barrier()


def _make_sc_scatter():
    mesh = plsc.VectorSubcoreMesh(
        core_axis_name="c", subcore_axis_name="s", num_cores=NC,
        num_subcores=NS)

    return pl.kernel(
        _sc_body,
        out_type=jax.ShapeDtypeStruct((NCHUNK * N, CW), jnp.float32),
        mesh=mesh,
        scratch_types=[
            pltpu.VMEM((ZR, CW), jnp.float32),
            pltpu.VMEM((NBATCH, BATCH), jnp.int32),
            pltpu.VMEM((NBATCH, BATCH), jnp.int32),
            pltpu.VMEM((BATCH, CW), jnp.float32),
            pltpu.VMEM((BATCH, CW), jnp.float32),
            pltpu.VMEM((BATCH, CW), jnp.float32),
            pltpu.VMEM((BATCH, CW), jnp.float32),
            pltpu.VMEM((BATCH, CW), jnp.float32),
            pltpu.VMEM_SHARED((ACCROWS, CW), jnp.float32),
            pltpu.SemaphoreType.DMA,
            pltpu.SemaphoreType.DMA,
            pltpu.SemaphoreType.DMA,
        ],
    )


_sc_scatter = _make_sc_scatter()


def _lin1_body(x_ref, w_ref, b_ref, out_ref):
    out_ref[...] = jnp.dot(x_ref[...], w_ref[...],
                           preferred_element_type=jnp.float32) + b_ref[...]


_lin1 = pl.pallas_call(
    _lin1_body,
    grid=(GRID,),
    in_specs=[
        pl.BlockSpec((R, D_IN), lambda r: (r, 0)),
        pl.BlockSpec((D_IN, D_H), lambda r: (0, 0)),
        pl.BlockSpec((1, D_H), lambda r: (0, 0)),
    ],
    out_specs=pl.BlockSpec((R, D_H), lambda r: (r, 0)),
    out_shape=jax.ShapeDtypeStruct((N, D_H), jnp.float32),
)


def _layer_body(agg_ref, x0_ref, w_ref, out_ref, *, beta):
    agg = jnp.concatenate([agg_ref[c] for c in range(NCHUNK)], axis=-1)
    hh = (1.0 - ALPHA) * agg + ALPHA * x0_ref[...]
    out_ref[...] = (1.0 - beta) * hh + beta * jnp.dot(
        hh, w_ref[...], preferred_element_type=jnp.float32)


def _make_layer(beta):
    return pl.pallas_call(
        functools.partial(_layer_body, beta=beta),
        grid=(GRID,),
        in_specs=[
            pl.BlockSpec((NCHUNK, R, CW), lambda r: (0, r, 0)),
            pl.BlockSpec((R, D_H), lambda r: (r, 0)),
            pl.BlockSpec((D_H, D_H), lambda r: (0, 0)),
        ],
        out_specs=pl.BlockSpec((R, D_H), lambda r: (r, 0)),
        out_shape=jax.ShapeDtypeStruct((N, D_H), jnp.float32),
    )


_layers = [_make_layer(float(math.log(THETA / (i + 1) + 1.0)))
           for i in range(L_LAYERS)]


def _fuse_body(h0_ref, h1_ref, h2_ref, h3_ref, wf_ref, bf_ref, w2_ref, b2_ref,
               out_ref):
    hs = [h0_ref, h1_ref, h2_ref, h3_ref]
    acc = jnp.broadcast_to(bf_ref[...], (R, D_H)).astype(jnp.float32)
    for i in range(L_LAYERS):
        acc = acc + jnp.dot(hs[i][...], wf_ref[i],
                            preferred_element_type=jnp.float32)
    out_ref[...] = jnp.dot(acc, w2_ref[...],
                           preferred_element_type=jnp.float32) + b2_ref[...]


_fuse = pl.pallas_call(
    _fuse_body,
    grid=(GRID,),
    in_specs=[
        pl.BlockSpec((R, D_H), lambda r: (r, 0)),
        pl.BlockSpec((R, D_H), lambda r: (r, 0)),
        pl.BlockSpec((R, D_H), lambda r: (r, 0)),
        pl.BlockSpec((R, D_H), lambda r: (r, 0)),
        pl.BlockSpec((L_LAYERS, D_H, D_H), lambda r: (0, 0, 0)),
        pl.BlockSpec((1, D_H), lambda r: (0, 0)),
        pl.BlockSpec((D_H, D_H), lambda r: (0, 0)),
        pl.BlockSpec((1, D_H), lambda r: (0, 0)),
    ],
    out_specs=pl.BlockSpec((R, D_H), lambda r: (r, 0)),
    out_shape=jax.ShapeDtypeStruct((N, D_H), jnp.float32),
)


def kernel(x, edge_index, w_lin1, b_lin1, conv_ws, w_fuse, b_fuse, w_lin2,
           b_lin2):
    src = edge_index[0]
    dst = edge_index[1]
    src4 = (src[None, :] * NCHUNK
            + jnp.arange(NCHUNK, dtype=jnp.int32)[:, None])
    src4 = src4.reshape(NCHUNK, NS, NBATCH, BATCH)
    dl = dst[None, :] - (jnp.arange(NC, dtype=jnp.int32) * NLOC)[:, None]
    trash = NLOC + (jnp.arange(E, dtype=jnp.int32) % 8)[None, :]
    dl = jnp.where((dl >= 0) & (dl < NLOC), dl, trash)
    dst2 = dl.reshape(NC, NS, NBATCH, BATCH)

    x0 = _lin1(x, w_lin1, b_lin1.reshape(1, D_H))
    h = x0
    feats = []
    for i in range(L_LAYERS):
        agg = _sc_scatter(h.reshape(NCHUNK * N, CW), src4, dst2)
        h = _layers[i](agg.reshape(NCHUNK, N, CW), x0, conv_ws[i])
        feats.append(h)

    return _fuse(feats[0], feats[1], feats[2], feats[3],
                 w_fuse.reshape(L_LAYERS, D_H, D_H),
                 b_fuse.reshape(1, D_H),
                 w_lin2, b_lin2.reshape(1, D_H))
